# baseline (device time: 70900 ns/iter reference)
import jax
import jax.numpy as jnp
from jax import lax
from jax.experimental import pallas as pl
from jax.experimental.pallas import tpu as pltpu

M, N, K = 2048, 2048, 1024
MB = M // 2
NC = 8
CW = N // NC
HW = CW // 2


def kernel(A, B):
    def body(a_ref, b_ref, out_ref, p_send, p_recv,
             send1, recv1, send2, recv2):
        my_x = lax.axis_index("x")
        my_y = lax.axis_index("y")
        x_nbr = (1 - my_x, my_y)
        y_nbr = (my_x, 1 - my_y)
        rows = pl.ds(my_y * MB, MB)

        barrier = pltpu.get_barrier_semaphore()
        pl.semaphore_signal(barrier, inc=1, device_id=x_nbr,
                            device_id_type=pl.DeviceIdType.MESH)
        pl.semaphore_signal(barrier, inc=1, device_id=y_nbr,
                            device_id_type=pl.DeviceIdType.MESH)
        pl.semaphore_wait(barrier, 2)

        a = a_ref[rows, :].astype(jnp.bfloat16)

        rdma1 = [None] * NC
        for j in range(NC):
            bj = b_ref[:, j * CW:(j + 1) * CW].astype(jnp.bfloat16)
            p = jnp.dot(a, bj, preferred_element_type=jnp.float32)
            p_send[j, :, :] = p.astype(jnp.bfloat16)
            r = pltpu.make_async_remote_copy(
                src_ref=p_send.at[j], dst_ref=p_recv.at[j],
                send_sem=send1.at[j], recv_sem=recv1.at[j],
                device_id=x_nbr, device_id_type=pl.DeviceIdType.MESH,
            )
            r.start()
            rdma1[j] = r

        rdma2 = [[None, None] for _ in range(NC)]
        for j in range(NC):
            rdma1[j].wait_recv()
            for h in range(2):
                lo = j * CW + h * HW
                cols = pl.ds(lo, HW)
                out_ref[rows, cols] = (p_send[j, :, h * HW:(h + 1) * HW] +
                                       p_recv[j, :, h * HW:(h + 1) * HW])
                r2 = pltpu.make_async_remote_copy(
                    src_ref=out_ref.at[rows, cols],
                    dst_ref=out_ref.at[rows, cols],
                    send_sem=send2.at[j, h], recv_sem=recv2.at[j, h],
                    device_id=y_nbr, device_id_type=pl.DeviceIdType.MESH,
                )
                r2.start()
                rdma2[j][h] = r2

        for j in range(NC):
            rdma2[j][0].wait_recv()
            rdma2[j][1].wait_recv()
            rdma1[j].wait_send()
            rdma2[j][0].wait_send()
            rdma2[j][1].wait_send()

    return pl.pallas_call(
        body,
        out_shape=jax.ShapeDtypeStruct((M, N), jnp.bfloat16),
        in_specs=[pl.BlockSpec(memory_space=pltpu.VMEM),
                  pl.BlockSpec(memory_space=pltpu.VMEM)],
        out_specs=pl.BlockSpec(memory_space=pltpu.VMEM),
        scratch_shapes=[
            pltpu.VMEM((NC, MB, CW), jnp.bfloat16),
            pltpu.VMEM((NC, MB, CW), jnp.bfloat16),
            pltpu.SemaphoreType.DMA((NC,)),
            pltpu.SemaphoreType.DMA((NC,)),
            pltpu.SemaphoreType.DMA((NC, 2)),
            pltpu.SemaphoreType.DMA((NC, 2)),
        ],
        compiler_params=pltpu.CompilerParams(collective_id=0),
    )(A, B)
